# Initial kernel scaffold; baseline (speedup 1.0000x reference)
import jax
import jax.numpy as jnp
from jax.experimental import pallas as pl


def kernel(x, edge_index, W0, b0, gamma0, beta0, W1, b1, gamma1, beta1, Wr0, br0, Wr1, br1):
    raise NotImplementedError("write your pallas kernel here")


# speedup vs baseline: 4.1257x; 4.1257x over previous
import functools

import jax
import jax.numpy as jnp
from jax import lax
from jax.experimental import pallas as pl
from jax.experimental.pallas import tpu as pltpu
import jax.experimental.pallas.tpu_sc as plsc

N = 10000
E = 320000
D_IN = 128
H0 = 256
H1 = 64
EPS = 1e-5

NC = 2
NS = 16
NW = NC * NS
PER_W = E // NW
CH = 80
NCHUNK = PER_W // CH
RPT = 632
NPAD = NS * RPT

R = 1000
GRID = N // R


def _mesh():
    return plsc.VectorSubcoreMesh(core_axis_name="c", subcore_axis_name="s")


@functools.cache
def _sc_deg():
    PER_T = E // NS
    NCH_T = PER_T // CH

    @functools.partial(
        pl.kernel,
        out_type=jax.ShapeDtypeStruct((NC, NPAD, D_IN), jnp.float32),
        mesh=_mesh(),
        scratch_types=[
            pltpu.VMEM((CH,), jnp.int32),
            pltpu.VMEM((CH, D_IN), jnp.float32),
            pltpu.VMEM_SHARED((NPAD, D_IN), jnp.float32),
        ],
    )
    def deg_kernel(src_hbm, dst_hbm, ones_hbm, zeros_hbm, deg_hbm,
                   idx_v, ones_v, acc):
        c = lax.axis_index("c")
        s = lax.axis_index("s")
        pltpu.sync_copy(ones_hbm, ones_v)
        pltpu.sync_copy(zeros_hbm, acc.at[pl.ds(s * RPT, RPT)])
        plsc.subcore_barrier()

        def body_src(j, carry):
            base = pl.multiple_of(s * PER_T + j * CH, 8)
            pltpu.sync_copy(src_hbm.at[pl.ds(base, CH)], idx_v)
            pltpu.sync_copy(ones_v, acc.at[idx_v], add=True)
            return carry

        def body_dst(j, carry):
            base = pl.multiple_of(s * PER_T + j * CH, 8)
            pltpu.sync_copy(dst_hbm.at[pl.ds(base, CH)], idx_v)
            pltpu.sync_copy(ones_v, acc.at[idx_v], add=True)
            return carry

        @pl.when(c == 0)
        def _():
            lax.fori_loop(0, NCH_T, body_src, 0)

        @pl.when(c == 1)
        def _():
            lax.fori_loop(0, NCH_T, body_dst, 0)

        plsc.subcore_barrier()
        pltpu.sync_copy(acc.at[pl.ds(s * RPT, RPT)],
                        deg_hbm.at[c, pl.ds(s * RPT, RPT)])

    return deg_kernel


@functools.cache
def _sc_agg(d):
    @functools.partial(
        pl.kernel,
        out_type=jax.ShapeDtypeStruct((NC, NPAD, d), jnp.float32),
        mesh=_mesh(),
        scratch_types=[
            pltpu.VMEM((CH,), jnp.int32),
            pltpu.VMEM((CH,), jnp.int32),
            pltpu.VMEM((CH, d), jnp.float32),
            pltpu.VMEM_SHARED((NPAD, d), jnp.float32),
            pltpu.SemaphoreType.DMA,
        ],
    )
    def agg_kernel(tab_hbm, src_hbm, dst_hbm, zeros_hbm, out_hbm,
                   sidx_v, didx_v, rows_v, acc, gsem):
        c = lax.axis_index("c")
        s = lax.axis_index("s")
        wid = s * NC + c
        pltpu.sync_copy(zeros_hbm, acc.at[pl.ds(s * RPT, RPT)])
        plsc.subcore_barrier()

        def body(j, carry):
            base = pl.multiple_of(wid * PER_W + j * CH, 8)
            pltpu.sync_copy(src_hbm.at[pl.ds(base, CH)], sidx_v)
            pltpu.sync_copy(dst_hbm.at[pl.ds(base, CH)], didx_v)
            pltpu.async_copy(tab_hbm.at[sidx_v], rows_v, gsem).wait()
            pltpu.sync_copy(rows_v, acc.at[didx_v], add=True)
            return carry

        lax.fori_loop(0, NCHUNK, body, 0)
        plsc.subcore_barrier()
        pltpu.sync_copy(acc.at[pl.ds(s * RPT, RPT)],
                        out_hbm.at[c, pl.ds(s * RPT, RPT)])

    return agg_kernel


def _tc_norms(deg, x):
    def body(deg_ref, x_ref, t0_ref, no_ref, ni_ref):
        do = deg_ref[0][:, :1]
        di = deg_ref[1][:, :1]
        no = jnp.where(do > 0, lax.rsqrt(do), 0.0)
        ni = jnp.where(di > 0, lax.rsqrt(di), 0.0)
        no_b = jnp.broadcast_to(no, (R, D_IN))
        ni_b = jnp.broadcast_to(ni, (R, D_IN))
        t0_ref[...] = x_ref[...] * no_b
        no_ref[...] = no_b
        ni_ref[...] = ni_b

    return pl.pallas_call(
        body,
        grid=(GRID,),
        in_specs=[pl.BlockSpec((2, R, D_IN), lambda i: (0, i, 0)),
                  pl.BlockSpec((R, D_IN), lambda i: (i, 0))],
        out_specs=[pl.BlockSpec((R, D_IN), lambda i: (i, 0))] * 3,
        out_shape=[jax.ShapeDtypeStruct((N, D_IN), jnp.float32)] * 3,
    )(deg, x)


def _tc_enc0(p0, ni_b, W0, b0):
    def body(p_ref, ni_ref, w_ref, b_ref, h_ref, st_ref):
        m = (p_ref[0] + p_ref[1]) * ni_ref[...]
        h = jnp.dot(m, w_ref[...], preferred_element_type=jnp.float32,
                    precision=lax.Precision.HIGHEST)
        h = jnp.maximum(h + b_ref[...], 0.0)
        h_ref[...] = h
        s1 = jnp.sum(h, 0, keepdims=True)
        s2 = jnp.sum(h * h, 0, keepdims=True)
        st_ref[...] = jnp.concatenate([s1, s2], 0)[None]

    return pl.pallas_call(
        body,
        grid=(GRID,),
        in_specs=[pl.BlockSpec((2, R, D_IN), lambda i: (0, i, 0)),
                  pl.BlockSpec((R, D_IN), lambda i: (i, 0)),
                  pl.BlockSpec((D_IN, H0), lambda i: (0, 0)),
                  pl.BlockSpec((1, H0), lambda i: (0, 0))],
        out_specs=[pl.BlockSpec((R, H0), lambda i: (i, 0)),
                   pl.BlockSpec((1, 2, H0), lambda i: (i, 0, 0))],
        out_shape=[jax.ShapeDtypeStruct((N, H0), jnp.float32),
                   jax.ShapeDtypeStruct((GRID, 2, H0), jnp.float32)],
    )(p0, ni_b, W0, b0)


def _tc_bn0_mm1(h0, st0, gamma0, beta0, W1, no_b):
    def body(h_ref, st_ref, g_ref, be_ref, w_ref, no_ref, t1_ref):
        st = jnp.sum(st_ref[...], 0)
        mean = st[0:1] / N
        var = st[1:2] / N - mean * mean
        hn = (h_ref[...] - mean) * lax.rsqrt(var + EPS) * g_ref[...] + be_ref[...]
        t1 = jnp.dot(hn, w_ref[...], preferred_element_type=jnp.float32,
                    precision=lax.Precision.HIGHEST)
        t1 = t1 * no_ref[...][:, :H1]
        t1_ref[...] = jnp.concatenate([t1, jnp.zeros((R, D_IN - H1), jnp.float32)], 1)

    return pl.pallas_call(
        body,
        grid=(GRID,),
        in_specs=[pl.BlockSpec((R, H0), lambda i: (i, 0)),
                  pl.BlockSpec((GRID, 2, H0), lambda i: (0, 0, 0)),
                  pl.BlockSpec((1, H0), lambda i: (0, 0)),
                  pl.BlockSpec((1, H0), lambda i: (0, 0)),
                  pl.BlockSpec((H0, H1), lambda i: (0, 0)),
                  pl.BlockSpec((R, D_IN), lambda i: (i, 0))],
        out_specs=pl.BlockSpec((R, D_IN), lambda i: (i, 0)),
        out_shape=jax.ShapeDtypeStruct((N, D_IN), jnp.float32),
    )(h0, st0, gamma0, beta0, W1, no_b)


def _tc_enc1(p1, ni_b, b1):
    def body(p_ref, ni_ref, b_ref, u_ref, st_ref):
        u = (p_ref[0] + p_ref[1])[:, :H1] * ni_ref[...][:, :H1]
        u = jnp.maximum(u + b_ref[...], 0.0)
        u_ref[...] = u
        s1 = jnp.sum(u, 0, keepdims=True)
        s2 = jnp.sum(u * u, 0, keepdims=True)
        st_ref[...] = jnp.concatenate([s1, s2], 0)[None]

    return pl.pallas_call(
        body,
        grid=(GRID,),
        in_specs=[pl.BlockSpec((2, R, D_IN), lambda i: (0, i, 0)),
                  pl.BlockSpec((R, D_IN), lambda i: (i, 0)),
                  pl.BlockSpec((1, H1), lambda i: (0, 0))],
        out_specs=[pl.BlockSpec((R, H1), lambda i: (i, 0)),
                   pl.BlockSpec((1, 2, H1), lambda i: (i, 0, 0))],
        out_shape=[jax.ShapeDtypeStruct((N, H1), jnp.float32),
                   jax.ShapeDtypeStruct((GRID, 2, H1), jnp.float32)],
    )(p1, ni_b, b1)


def _tc_bn1(u, st1, gamma1, beta1, no_b):
    def body(u_ref, st_ref, g_ref, be_ref, no_ref, h2_ref, t2_ref):
        st = jnp.sum(st_ref[...], 0)
        mean = st[0:1] / N
        var = st[1:2] / N - mean * mean
        h2 = (u_ref[...] - mean) * lax.rsqrt(var + EPS) * g_ref[...] + be_ref[...]
        h2_ref[...] = h2
        t2 = h2 * no_ref[...][:, :H1]
        t2_ref[...] = jnp.concatenate([t2, jnp.zeros((R, D_IN - H1), jnp.float32)], 1)

    return pl.pallas_call(
        body,
        grid=(GRID,),
        in_specs=[pl.BlockSpec((R, H1), lambda i: (i, 0)),
                  pl.BlockSpec((GRID, 2, H1), lambda i: (0, 0, 0)),
                  pl.BlockSpec((1, H1), lambda i: (0, 0)),
                  pl.BlockSpec((1, H1), lambda i: (0, 0)),
                  pl.BlockSpec((R, D_IN), lambda i: (i, 0))],
        out_specs=[pl.BlockSpec((R, H1), lambda i: (i, 0)),
                   pl.BlockSpec((R, D_IN), lambda i: (i, 0))],
        out_shape=[jax.ShapeDtypeStruct((N, H1), jnp.float32),
                   jax.ShapeDtypeStruct((N, D_IN), jnp.float32)],
    )(u, st1, gamma1, beta1, no_b)


def _tc_adj(h2):
    def body(a_ref, b_ref, o_ref):
        p = lax.dot_general(a_ref[...], b_ref[...],
                            (((1,), (1,)), ((), ())),
                            preferred_element_type=jnp.float32,
                    precision=lax.Precision.HIGHEST)
        o_ref[...] = 1.0 / (1.0 + jnp.exp(-p))

    RA = 400
    return pl.pallas_call(
        body,
        grid=(N // RA,),
        in_specs=[pl.BlockSpec((RA, H1), lambda i: (i, 0)),
                  pl.BlockSpec((N, H1), lambda i: (0, 0))],
        out_specs=pl.BlockSpec((RA, N), lambda i: (i, 0)),
        out_shape=jax.ShapeDtypeStruct((N, N), jnp.float32),
    )(h2, h2)


def _tc_rec0(p2, ni_b, Wr0, br0, Wr1, no_b):
    def body(p_ref, ni_ref, w0_ref, b0_ref, w1_ref, no_ref, t3_ref):
        m = (p_ref[0] + p_ref[1])[:, :H1] * ni_ref[...][:, :H1]
        hr = jnp.dot(m, w0_ref[...], preferred_element_type=jnp.float32,
                    precision=lax.Precision.HIGHEST)
        hr = jnp.maximum(hr + b0_ref[...], 0.0)
        t3 = jnp.dot(hr, w1_ref[...], preferred_element_type=jnp.float32,
                    precision=lax.Precision.HIGHEST)
        t3_ref[...] = t3 * no_ref[...]

    return pl.pallas_call(
        body,
        grid=(GRID,),
        in_specs=[pl.BlockSpec((2, R, D_IN), lambda i: (0, i, 0)),
                  pl.BlockSpec((R, D_IN), lambda i: (i, 0)),
                  pl.BlockSpec((H1, H0), lambda i: (0, 0)),
                  pl.BlockSpec((1, H0), lambda i: (0, 0)),
                  pl.BlockSpec((H0, D_IN), lambda i: (0, 0)),
                  pl.BlockSpec((R, D_IN), lambda i: (i, 0))],
        out_specs=pl.BlockSpec((R, D_IN), lambda i: (i, 0)),
        out_shape=jax.ShapeDtypeStruct((N, D_IN), jnp.float32),
    )(p2, ni_b, Wr0, br0, Wr1, no_b)


def _tc_rec1(p3, ni_b, br1):
    def body(p_ref, ni_ref, b_ref, o_ref):
        o_ref[...] = (p_ref[0] + p_ref[1]) * ni_ref[...] + b_ref[...]

    return pl.pallas_call(
        body,
        grid=(GRID,),
        in_specs=[pl.BlockSpec((2, R, D_IN), lambda i: (0, i, 0)),
                  pl.BlockSpec((R, D_IN), lambda i: (i, 0)),
                  pl.BlockSpec((1, D_IN), lambda i: (0, 0))],
        out_specs=pl.BlockSpec((R, D_IN), lambda i: (i, 0)),
        out_shape=jax.ShapeDtypeStruct((N, D_IN), jnp.float32),
    )(p3, ni_b, br1)


def kernel(x, edge_index, W0, b0, gamma0, beta0, W1, b1, gamma1, beta1,
           Wr0, br0, Wr1, br1):
    src = edge_index[0].astype(jnp.int32)
    dst = edge_index[1].astype(jnp.int32)
    ones128 = jnp.ones((CH, D_IN), jnp.float32)
    z128 = jnp.zeros((RPT, D_IN), jnp.float32)

    deg = _sc_deg()(src, dst, ones128, z128)
    t0, no_b, ni_b = _tc_norms(deg, x)

    p0 = _sc_agg(D_IN)(t0, src, dst, z128)
    h0, st0 = _tc_enc0(p0, ni_b, W0, b0.reshape(1, H0))
    t1 = _tc_bn0_mm1(h0, st0, gamma0.reshape(1, H0), beta0.reshape(1, H0),
                     W1, no_b)

    p1 = _sc_agg(D_IN)(t1, src, dst, z128)
    u, st1 = _tc_enc1(p1, ni_b, b1.reshape(1, H1))
    h2, t2 = _tc_bn1(u, st1, gamma1.reshape(1, H1), beta1.reshape(1, H1),
                     no_b)

    adj = _tc_adj(h2)

    p2 = _sc_agg(D_IN)(t2, src, dst, z128)
    t3 = _tc_rec0(p2, ni_b, Wr0, br0.reshape(1, H0), Wr1, no_b)

    p3 = _sc_agg(D_IN)(t3, src, dst, z128)
    h_out = _tc_rec1(p3, ni_b, br1.reshape(1, D_IN))

    return adj, h_out


# speedup vs baseline: 6.2502x; 1.5149x over previous
import functools

import jax
import jax.numpy as jnp
from jax import lax
from jax.experimental import pallas as pl
from jax.experimental.pallas import tpu as pltpu
import jax.experimental.pallas.tpu_sc as plsc

N = 10000
E = 320000
D_IN = 128
H0 = 256
H1 = 64
EPS = 1e-5

NC = 2
NS = 16
NW = NC * NS
PER_W = E // NW
CH = 80
NCHUNK = PER_W // CH
CH2 = 128
NCHUNK2 = 79
E2 = NW * NCHUNK2 * CH2
PER_W2 = E2 // NW
NTRASH = 96
RPT = 632
NPAD = NS * RPT

R = 1000
GRID = N // R


def _mesh():
    return plsc.VectorSubcoreMesh(core_axis_name="c", subcore_axis_name="s")


@functools.cache
def _sc_deg():
    PER_T = E // NS
    NCH_T = PER_T // CH

    @functools.partial(
        pl.kernel,
        out_type=jax.ShapeDtypeStruct((NC, NPAD, D_IN), jnp.float32),
        mesh=_mesh(),
        scratch_types=[
            pltpu.VMEM((CH,), jnp.int32),
            pltpu.VMEM((CH, D_IN), jnp.float32),
            pltpu.VMEM_SHARED((NPAD, D_IN), jnp.float32),
        ],
    )
    def deg_kernel(src_hbm, dst_hbm, ones_hbm, zeros_hbm, deg_hbm,
                   idx_v, ones_v, acc):
        c = lax.axis_index("c")
        s = lax.axis_index("s")
        pltpu.sync_copy(ones_hbm, ones_v)
        pltpu.sync_copy(zeros_hbm, acc.at[pl.ds(s * RPT, RPT)])
        plsc.subcore_barrier()

        def body_src(j, carry):
            base = pl.multiple_of(s * PER_T + j * CH, 8)
            pltpu.sync_copy(src_hbm.at[pl.ds(base, CH)], idx_v)
            pltpu.sync_copy(ones_v, acc.at[idx_v], add=True)
            return carry

        def body_dst(j, carry):
            base = pl.multiple_of(s * PER_T + j * CH, 8)
            pltpu.sync_copy(dst_hbm.at[pl.ds(base, CH)], idx_v)
            pltpu.sync_copy(ones_v, acc.at[idx_v], add=True)
            return carry

        @pl.when(c == 0)
        def _():
            lax.fori_loop(0, NCH_T, body_src, 0)

        @pl.when(c == 1)
        def _():
            lax.fori_loop(0, NCH_T, body_dst, 0)

        plsc.subcore_barrier()
        pltpu.sync_copy(acc.at[pl.ds(s * RPT, RPT)],
                        deg_hbm.at[c, pl.ds(s * RPT, RPT)])

    return deg_kernel


@functools.cache
def _sc_agg(d):
    @functools.partial(
        pl.kernel,
        out_type=jax.ShapeDtypeStruct((NC, NPAD, d), jnp.float32),
        mesh=_mesh(),
        scratch_types=[
            pltpu.VMEM((2, CH2), jnp.int32),
            pltpu.VMEM((2, CH2), jnp.int32),
            pltpu.VMEM((2, CH2, d), jnp.float32),
            pltpu.VMEM_SHARED((NPAD, d), jnp.float32),
            pltpu.SemaphoreType.DMA((2,)),
            pltpu.SemaphoreType.DMA((2,)),
            pltpu.SemaphoreType.DMA((2,)),
            pltpu.SemaphoreType.DMA((2,)),
        ],
    )
    def agg_kernel(tab_hbm, src_hbm, dst_hbm, zeros_hbm, out_hbm,
                   sidx, didx, rows, acc, semis, semid, semg, sems):
        c = lax.axis_index("c")
        s = lax.axis_index("s")
        wid = s * NC + c
        base0 = wid * PER_W2
        pltpu.sync_copy(zeros_hbm, acc.at[pl.ds(s * RPT, RPT)])
        plsc.subcore_barrier()

        def body(j, carry):
            b = lax.rem(j, 2)
            o = 1 - b

            @pl.when(j >= 2)
            def _():
                pltpu.make_async_copy(rows.at[b], acc.at[didx.at[b]],
                                      sems.at[b]).wait()

            base = pl.multiple_of(base0 + j * CH2, 8)
            pltpu.async_copy(src_hbm.at[pl.ds(base, CH2)], sidx.at[b],
                             semis.at[b])
            pltpu.async_copy(dst_hbm.at[pl.ds(base, CH2)], didx.at[b],
                             semid.at[b])

            @pl.when(j >= 1)
            def _():
                pltpu.make_async_copy(tab_hbm.at[sidx.at[o]], rows.at[o],
                                      semg.at[o]).wait()
                pltpu.async_copy(rows.at[o], acc.at[didx.at[o]], sems.at[o],
                                 add=True)

            pltpu.make_async_copy(src_hbm.at[pl.ds(base, CH2)], sidx.at[b],
                                  semis.at[b]).wait()
            pltpu.make_async_copy(dst_hbm.at[pl.ds(base, CH2)], didx.at[b],
                                  semid.at[b]).wait()
            pltpu.async_copy(tab_hbm.at[sidx.at[b]], rows.at[b], semg.at[b])
            return carry

        lax.fori_loop(0, NCHUNK2, body, 0)

        bl = (NCHUNK2 - 1) % 2
        pltpu.make_async_copy(tab_hbm.at[sidx.at[bl]], rows.at[bl],
                              semg.at[bl]).wait()
        pltpu.async_copy(rows.at[bl], acc.at[didx.at[bl]], sems.at[bl],
                         add=True)
        pltpu.make_async_copy(rows.at[1 - bl], acc.at[didx.at[1 - bl]],
                              sems.at[1 - bl]).wait()
        pltpu.make_async_copy(rows.at[bl], acc.at[didx.at[bl]],
                              sems.at[bl]).wait()
        plsc.subcore_barrier()
        pltpu.sync_copy(acc.at[pl.ds(s * RPT, RPT)],
                        out_hbm.at[c, pl.ds(s * RPT, RPT)])

    return agg_kernel


def _tc_norms(deg, x):
    def body(deg_ref, x_ref, t0_ref, no_ref, ni_ref):
        do = deg_ref[0][:, :1]
        di = deg_ref[1][:, :1]
        no = jnp.where(do > 0, lax.rsqrt(do), 0.0)
        ni = jnp.where(di > 0, lax.rsqrt(di), 0.0)
        no_b = jnp.broadcast_to(no, (R, D_IN))
        ni_b = jnp.broadcast_to(ni, (R, D_IN))
        t0_ref[...] = x_ref[...] * no_b
        no_ref[...] = no_b
        ni_ref[...] = ni_b

    return pl.pallas_call(
        body,
        grid=(GRID,),
        in_specs=[pl.BlockSpec((2, R, D_IN), lambda i: (0, i, 0)),
                  pl.BlockSpec((R, D_IN), lambda i: (i, 0))],
        out_specs=[pl.BlockSpec((R, D_IN), lambda i: (i, 0))] * 3,
        out_shape=[jax.ShapeDtypeStruct((N, D_IN), jnp.float32)] * 3,
    )(deg, x)


def _tc_enc0(p0, ni_b, W0, b0):
    def body(p_ref, ni_ref, w_ref, b_ref, h_ref, st_ref):
        m = (p_ref[0] + p_ref[1]) * ni_ref[...]
        h = jnp.dot(m, w_ref[...], preferred_element_type=jnp.float32,
                    precision=lax.Precision.HIGHEST)
        h = jnp.maximum(h + b_ref[...], 0.0)
        h_ref[...] = h
        s1 = jnp.sum(h, 0, keepdims=True)
        s2 = jnp.sum(h * h, 0, keepdims=True)
        st_ref[...] = jnp.concatenate([s1, s2], 0)[None]

    return pl.pallas_call(
        body,
        grid=(GRID,),
        in_specs=[pl.BlockSpec((2, R, D_IN), lambda i: (0, i, 0)),
                  pl.BlockSpec((R, D_IN), lambda i: (i, 0)),
                  pl.BlockSpec((D_IN, H0), lambda i: (0, 0)),
                  pl.BlockSpec((1, H0), lambda i: (0, 0))],
        out_specs=[pl.BlockSpec((R, H0), lambda i: (i, 0)),
                   pl.BlockSpec((1, 2, H0), lambda i: (i, 0, 0))],
        out_shape=[jax.ShapeDtypeStruct((N, H0), jnp.float32),
                   jax.ShapeDtypeStruct((GRID, 2, H0), jnp.float32)],
    )(p0, ni_b, W0, b0)


def _tc_bn0_mm1(h0, st0, gamma0, beta0, W1, no_b):
    def body(h_ref, st_ref, g_ref, be_ref, w_ref, no_ref, t1_ref):
        st = jnp.sum(st_ref[...], 0)
        mean = st[0:1] / N
        var = st[1:2] / N - mean * mean
        hn = (h_ref[...] - mean) * lax.rsqrt(var + EPS) * g_ref[...] + be_ref[...]
        t1 = jnp.dot(hn, w_ref[...], preferred_element_type=jnp.float32,
                    precision=lax.Precision.HIGHEST)
        t1 = t1 * no_ref[...][:, :H1]
        t1_ref[...] = jnp.concatenate([t1, jnp.zeros((R, D_IN - H1), jnp.float32)], 1)

    return pl.pallas_call(
        body,
        grid=(GRID,),
        in_specs=[pl.BlockSpec((R, H0), lambda i: (i, 0)),
                  pl.BlockSpec((GRID, 2, H0), lambda i: (0, 0, 0)),
                  pl.BlockSpec((1, H0), lambda i: (0, 0)),
                  pl.BlockSpec((1, H0), lambda i: (0, 0)),
                  pl.BlockSpec((H0, H1), lambda i: (0, 0)),
                  pl.BlockSpec((R, D_IN), lambda i: (i, 0))],
        out_specs=pl.BlockSpec((R, D_IN), lambda i: (i, 0)),
        out_shape=jax.ShapeDtypeStruct((N, D_IN), jnp.float32),
    )(h0, st0, gamma0, beta0, W1, no_b)


def _tc_enc1(p1, ni_b, b1):
    def body(p_ref, ni_ref, b_ref, u_ref, st_ref):
        u = (p_ref[0] + p_ref[1])[:, :H1] * ni_ref[...][:, :H1]
        u = jnp.maximum(u + b_ref[...], 0.0)
        u_ref[...] = u
        s1 = jnp.sum(u, 0, keepdims=True)
        s2 = jnp.sum(u * u, 0, keepdims=True)
        st_ref[...] = jnp.concatenate([s1, s2], 0)[None]

    return pl.pallas_call(
        body,
        grid=(GRID,),
        in_specs=[pl.BlockSpec((2, R, D_IN), lambda i: (0, i, 0)),
                  pl.BlockSpec((R, D_IN), lambda i: (i, 0)),
                  pl.BlockSpec((1, H1), lambda i: (0, 0))],
        out_specs=[pl.BlockSpec((R, H1), lambda i: (i, 0)),
                   pl.BlockSpec((1, 2, H1), lambda i: (i, 0, 0))],
        out_shape=[jax.ShapeDtypeStruct((N, H1), jnp.float32),
                   jax.ShapeDtypeStruct((GRID, 2, H1), jnp.float32)],
    )(p1, ni_b, b1)


def _tc_bn1(u, st1, gamma1, beta1, no_b):
    def body(u_ref, st_ref, g_ref, be_ref, no_ref, h2_ref, t2_ref):
        st = jnp.sum(st_ref[...], 0)
        mean = st[0:1] / N
        var = st[1:2] / N - mean * mean
        h2 = (u_ref[...] - mean) * lax.rsqrt(var + EPS) * g_ref[...] + be_ref[...]
        h2_ref[...] = h2
        t2 = h2 * no_ref[...][:, :H1]
        t2_ref[...] = jnp.concatenate([t2, jnp.zeros((R, D_IN - H1), jnp.float32)], 1)

    return pl.pallas_call(
        body,
        grid=(GRID,),
        in_specs=[pl.BlockSpec((R, H1), lambda i: (i, 0)),
                  pl.BlockSpec((GRID, 2, H1), lambda i: (0, 0, 0)),
                  pl.BlockSpec((1, H1), lambda i: (0, 0)),
                  pl.BlockSpec((1, H1), lambda i: (0, 0)),
                  pl.BlockSpec((R, D_IN), lambda i: (i, 0))],
        out_specs=[pl.BlockSpec((R, H1), lambda i: (i, 0)),
                   pl.BlockSpec((R, D_IN), lambda i: (i, 0))],
        out_shape=[jax.ShapeDtypeStruct((N, H1), jnp.float32),
                   jax.ShapeDtypeStruct((N, D_IN), jnp.float32)],
    )(u, st1, gamma1, beta1, no_b)


def _tc_adj(h2):
    def body(a_ref, b_ref, o_ref):
        p = lax.dot_general(a_ref[...], b_ref[...],
                            (((1,), (1,)), ((), ())),
                            preferred_element_type=jnp.float32,
                    precision=lax.Precision.HIGHEST)
        o_ref[...] = 1.0 / (1.0 + jnp.exp(-p))

    RA = 400
    return pl.pallas_call(
        body,
        grid=(N // RA,),
        in_specs=[pl.BlockSpec((RA, H1), lambda i: (i, 0)),
                  pl.BlockSpec((N, H1), lambda i: (0, 0))],
        out_specs=pl.BlockSpec((RA, N), lambda i: (i, 0)),
        out_shape=jax.ShapeDtypeStruct((N, N), jnp.float32),
    )(h2, h2)


def _tc_rec0(p2, ni_b, Wr0, br0, Wr1, no_b):
    def body(p_ref, ni_ref, w0_ref, b0_ref, w1_ref, no_ref, t3_ref):
        m = (p_ref[0] + p_ref[1])[:, :H1] * ni_ref[...][:, :H1]
        hr = jnp.dot(m, w0_ref[...], preferred_element_type=jnp.float32,
                    precision=lax.Precision.HIGHEST)
        hr = jnp.maximum(hr + b0_ref[...], 0.0)
        t3 = jnp.dot(hr, w1_ref[...], preferred_element_type=jnp.float32,
                    precision=lax.Precision.HIGHEST)
        t3_ref[...] = t3 * no_ref[...]

    return pl.pallas_call(
        body,
        grid=(GRID,),
        in_specs=[pl.BlockSpec((2, R, D_IN), lambda i: (0, i, 0)),
                  pl.BlockSpec((R, D_IN), lambda i: (i, 0)),
                  pl.BlockSpec((H1, H0), lambda i: (0, 0)),
                  pl.BlockSpec((1, H0), lambda i: (0, 0)),
                  pl.BlockSpec((H0, D_IN), lambda i: (0, 0)),
                  pl.BlockSpec((R, D_IN), lambda i: (i, 0))],
        out_specs=pl.BlockSpec((R, D_IN), lambda i: (i, 0)),
        out_shape=jax.ShapeDtypeStruct((N, D_IN), jnp.float32),
    )(p2, ni_b, Wr0, br0, Wr1, no_b)


def _tc_rec1(p3, ni_b, br1):
    def body(p_ref, ni_ref, b_ref, o_ref):
        o_ref[...] = (p_ref[0] + p_ref[1]) * ni_ref[...] + b_ref[...]

    return pl.pallas_call(
        body,
        grid=(GRID,),
        in_specs=[pl.BlockSpec((2, R, D_IN), lambda i: (0, i, 0)),
                  pl.BlockSpec((R, D_IN), lambda i: (i, 0)),
                  pl.BlockSpec((1, D_IN), lambda i: (0, 0))],
        out_specs=pl.BlockSpec((R, D_IN), lambda i: (i, 0)),
        out_shape=jax.ShapeDtypeStruct((N, D_IN), jnp.float32),
    )(p3, ni_b, br1)


def kernel(x, edge_index, W0, b0, gamma0, beta0, W1, b1, gamma1, beta1,
           Wr0, br0, Wr1, br1):
    src = edge_index[0].astype(jnp.int32)
    dst = edge_index[1].astype(jnp.int32)
    npad_e = E2 - E
    src2 = jnp.concatenate([src, jnp.arange(npad_e, dtype=jnp.int32) % N])
    dst2 = jnp.concatenate(
        [dst, N + jnp.arange(npad_e, dtype=jnp.int32) % NTRASH])
    ones128 = jnp.ones((CH, D_IN), jnp.float32)
    z128 = jnp.zeros((RPT, D_IN), jnp.float32)

    deg = _sc_deg()(src, dst, ones128, z128)
    t0, no_b, ni_b = _tc_norms(deg, x)

    p0 = _sc_agg(D_IN)(t0, src2, dst2, z128)
    h0, st0 = _tc_enc0(p0, ni_b, W0, b0.reshape(1, H0))
    t1 = _tc_bn0_mm1(h0, st0, gamma0.reshape(1, H0), beta0.reshape(1, H0),
                     W1, no_b)

    p1 = _sc_agg(D_IN)(t1, src2, dst2, z128)
    u, st1 = _tc_enc1(p1, ni_b, b1.reshape(1, H1))
    h2, t2 = _tc_bn1(u, st1, gamma1.reshape(1, H1), beta1.reshape(1, H1),
                     no_b)

    adj = _tc_adj(h2)

    p2 = _sc_agg(D_IN)(t2, src2, dst2, z128)
    t3 = _tc_rec0(p2, ni_b, Wr0, br0.reshape(1, H0), Wr1, no_b)

    p3 = _sc_agg(D_IN)(t3, src2, dst2, z128)
    h_out = _tc_rec1(p3, ni_b, br1.reshape(1, D_IN))

    return adj, h_out


# speedup vs baseline: 7.1762x; 1.1482x over previous
import functools

import jax
import jax.numpy as jnp
from jax import lax
from jax.experimental import pallas as pl
from jax.experimental.pallas import tpu as pltpu
import jax.experimental.pallas.tpu_sc as plsc

N = 10000
E = 320000
D_IN = 128
H0 = 256
H1 = 64
EPS = 1e-5

NC = 2
NS = 16
NW = NC * NS
PER_W = E // NW
CH = 80
NCHUNK = PER_W // CH
CH2 = 128
NCHUNK2 = 79
E2 = NW * NCHUNK2 * CH2
PER_W2 = E2 // NW
NTRASH = 96
RPT = 640
NPAD = NS * RPT

R = 1000
GRID = N // R


def _mesh():
    return plsc.VectorSubcoreMesh(core_axis_name="c", subcore_axis_name="s")


@functools.cache
def _sc_deg():
    PER_T = E2 // NS
    NCH_T = PER_T // CH2

    @functools.partial(
        pl.kernel,
        out_type=jax.ShapeDtypeStruct((NC * NPAD,), jnp.float32),
        mesh=_mesh(),
        scratch_types=[
            pltpu.VMEM((2, CH2), jnp.int32),
            pltpu.VMEM((CH2,), jnp.float32),
            pltpu.VMEM((RPT,), jnp.float32),
            pltpu.VMEM_SHARED((NPAD,), jnp.float32),
            pltpu.SemaphoreType.DMA((2,)),
            pltpu.SemaphoreType.DMA((2,)),
        ],
    )
    def deg_kernel(src_hbm, dst_hbm, deg_hbm,
                   idx, ones_v, zbuf, acc, semi, sems):
        c = lax.axis_index("c")
        s = lax.axis_index("s")
        base0 = s * PER_T

        def fill(i, carry):
            ones_v[pl.ds(i * 16, 16)] = jnp.full((16,), 1.0, jnp.float32)
            return carry

        lax.fori_loop(0, CH2 // 16, fill, 0)

        def fill_zero(i, carry):
            zbuf[pl.ds(i * 16, 16)] = jnp.zeros((16,), jnp.float32)
            return carry

        lax.fori_loop(0, RPT // 16, fill_zero, 0)
        pltpu.sync_copy(zbuf, acc.at[pl.ds(s * RPT, RPT)])
        plsc.subcore_barrier()

        def body(j, carry):
            b = lax.rem(j, 2)

            @pl.when(j >= 2)
            def _():
                pltpu.make_async_copy(ones_v, acc.at[idx.at[b]],
                                      sems.at[b]).wait()

            base = pl.multiple_of(base0 + j * CH2, 8)

            @pl.when(c == 0)
            def _():
                pltpu.async_copy(src_hbm.at[pl.ds(base, CH2)], idx.at[b],
                                 semi.at[b])

            @pl.when(c == 1)
            def _():
                pltpu.async_copy(dst_hbm.at[pl.ds(base, CH2)], idx.at[b],
                                 semi.at[b])

            pltpu.make_async_copy(src_hbm.at[pl.ds(base, CH2)], idx.at[b],
                                  semi.at[b]).wait()
            pltpu.async_copy(ones_v, acc.at[idx.at[b]], sems.at[b], add=True)
            return carry

        lax.fori_loop(0, NCH_T, body, 0)
        pltpu.make_async_copy(ones_v, acc.at[idx.at[0]], sems.at[0]).wait()
        pltpu.make_async_copy(ones_v, acc.at[idx.at[1]], sems.at[1]).wait()
        plsc.subcore_barrier()
        pltpu.sync_copy(acc.at[pl.ds(s * RPT, RPT)],
                        deg_hbm.at[pl.ds(c * NPAD + s * RPT, RPT)])

    return deg_kernel


@functools.cache
def _sc_agg(d):
    @functools.partial(
        pl.kernel,
        out_type=jax.ShapeDtypeStruct((NC, NPAD, d), jnp.float32),
        mesh=_mesh(),
        scratch_types=[
            pltpu.VMEM((2, CH2), jnp.int32),
            pltpu.VMEM((2, CH2), jnp.int32),
            pltpu.VMEM((2, CH2, d), jnp.float32),
            pltpu.VMEM_SHARED((NPAD, d), jnp.float32),
            pltpu.SemaphoreType.DMA((2,)),
            pltpu.SemaphoreType.DMA((2,)),
            pltpu.SemaphoreType.DMA((2,)),
            pltpu.SemaphoreType.DMA((2,)),
        ],
    )
    def agg_kernel(tab_hbm, src_hbm, dst_hbm, zeros_hbm, out_hbm,
                   sidx, didx, rows, acc, semis, semid, semg, sems):
        c = lax.axis_index("c")
        s = lax.axis_index("s")
        wid = s * NC + c
        base0 = wid * PER_W2
        pltpu.sync_copy(zeros_hbm, acc.at[pl.ds(s * RPT, RPT)])
        plsc.subcore_barrier()

        def body(j, carry):
            b = lax.rem(j, 2)
            o = 1 - b

            @pl.when(j >= 2)
            def _():
                pltpu.make_async_copy(rows.at[b], acc.at[didx.at[b]],
                                      sems.at[b]).wait()

            base = pl.multiple_of(base0 + j * CH2, 8)
            pltpu.async_copy(src_hbm.at[pl.ds(base, CH2)], sidx.at[b],
                             semis.at[b])
            pltpu.async_copy(dst_hbm.at[pl.ds(base, CH2)], didx.at[b],
                             semid.at[b])

            @pl.when(j >= 1)
            def _():
                pltpu.make_async_copy(tab_hbm.at[sidx.at[o]], rows.at[o],
                                      semg.at[o]).wait()
                pltpu.async_copy(rows.at[o], acc.at[didx.at[o]], sems.at[o],
                                 add=True)

            pltpu.make_async_copy(src_hbm.at[pl.ds(base, CH2)], sidx.at[b],
                                  semis.at[b]).wait()
            pltpu.make_async_copy(dst_hbm.at[pl.ds(base, CH2)], didx.at[b],
                                  semid.at[b]).wait()
            pltpu.async_copy(tab_hbm.at[sidx.at[b]], rows.at[b], semg.at[b])
            return carry

        lax.fori_loop(0, NCHUNK2, body, 0)

        bl = (NCHUNK2 - 1) % 2
        pltpu.make_async_copy(tab_hbm.at[sidx.at[bl]], rows.at[bl],
                              semg.at[bl]).wait()
        pltpu.async_copy(rows.at[bl], acc.at[didx.at[bl]], sems.at[bl],
                         add=True)
        pltpu.make_async_copy(rows.at[1 - bl], acc.at[didx.at[1 - bl]],
                              sems.at[1 - bl]).wait()
        pltpu.make_async_copy(rows.at[bl], acc.at[didx.at[bl]],
                              sems.at[bl]).wait()
        plsc.subcore_barrier()
        pltpu.sync_copy(acc.at[pl.ds(s * RPT, RPT)],
                        out_hbm.at[c, pl.ds(s * RPT, RPT)])

    return agg_kernel


def _tc_norms(deg, x):
    def body(deg_ref, x_ref, t0_ref, no_ref, ni_ref):
        do = deg_ref[0]
        di = deg_ref[1]
        no = jnp.where(do > 0, lax.rsqrt(do), 0.0)
        ni = jnp.where(di > 0, lax.rsqrt(di), 0.0)
        no_b = jnp.broadcast_to(no, (R, D_IN))
        ni_b = jnp.broadcast_to(ni, (R, D_IN))
        t0_ref[...] = x_ref[...] * no_b
        no_ref[...] = no_b
        ni_ref[...] = ni_b

    return pl.pallas_call(
        body,
        grid=(GRID,),
        in_specs=[pl.BlockSpec((2, R, 1), lambda i: (0, i, 0)),
                  pl.BlockSpec((R, D_IN), lambda i: (i, 0))],
        out_specs=[pl.BlockSpec((R, D_IN), lambda i: (i, 0))] * 3,
        out_shape=[jax.ShapeDtypeStruct((N, D_IN), jnp.float32)] * 3,
    )(deg, x)


def _tc_enc0(p0, ni_b, W0, b0):
    def body(p_ref, ni_ref, w_ref, b_ref, h_ref, st_ref):
        m = (p_ref[0] + p_ref[1]) * ni_ref[...]
        h = jnp.dot(m, w_ref[...], preferred_element_type=jnp.float32,
                    precision=lax.Precision.HIGHEST)
        h = jnp.maximum(h + b_ref[...], 0.0)
        h_ref[...] = h
        s1 = jnp.sum(h, 0, keepdims=True)
        s2 = jnp.sum(h * h, 0, keepdims=True)
        st_ref[...] = jnp.concatenate([s1, s2], 0)[None]

    return pl.pallas_call(
        body,
        grid=(GRID,),
        in_specs=[pl.BlockSpec((2, R, D_IN), lambda i: (0, i, 0)),
                  pl.BlockSpec((R, D_IN), lambda i: (i, 0)),
                  pl.BlockSpec((D_IN, H0), lambda i: (0, 0)),
                  pl.BlockSpec((1, H0), lambda i: (0, 0))],
        out_specs=[pl.BlockSpec((R, H0), lambda i: (i, 0)),
                   pl.BlockSpec((1, 2, H0), lambda i: (i, 0, 0))],
        out_shape=[jax.ShapeDtypeStruct((N, H0), jnp.float32),
                   jax.ShapeDtypeStruct((GRID, 2, H0), jnp.float32)],
    )(p0, ni_b, W0, b0)


def _tc_bn0_mm1(h0, st0, gamma0, beta0, W1, no_b):
    def body(h_ref, st_ref, g_ref, be_ref, w_ref, no_ref, t1_ref):
        st = jnp.sum(st_ref[...], 0)
        mean = st[0:1] / N
        var = st[1:2] / N - mean * mean
        hn = (h_ref[...] - mean) * lax.rsqrt(var + EPS) * g_ref[...] + be_ref[...]
        t1 = jnp.dot(hn, w_ref[...], preferred_element_type=jnp.float32,
                    precision=lax.Precision.HIGHEST)
        t1 = t1 * no_ref[...][:, :H1]
        t1_ref[...] = jnp.concatenate([t1, jnp.zeros((R, D_IN - H1), jnp.float32)], 1)

    return pl.pallas_call(
        body,
        grid=(GRID,),
        in_specs=[pl.BlockSpec((R, H0), lambda i: (i, 0)),
                  pl.BlockSpec((GRID, 2, H0), lambda i: (0, 0, 0)),
                  pl.BlockSpec((1, H0), lambda i: (0, 0)),
                  pl.BlockSpec((1, H0), lambda i: (0, 0)),
                  pl.BlockSpec((H0, H1), lambda i: (0, 0)),
                  pl.BlockSpec((R, D_IN), lambda i: (i, 0))],
        out_specs=pl.BlockSpec((R, D_IN), lambda i: (i, 0)),
        out_shape=jax.ShapeDtypeStruct((N, D_IN), jnp.float32),
    )(h0, st0, gamma0, beta0, W1, no_b)


def _tc_enc1(p1, ni_b, b1):
    def body(p_ref, ni_ref, b_ref, u_ref, st_ref):
        u = (p_ref[0] + p_ref[1])[:, :H1] * ni_ref[...][:, :H1]
        u = jnp.maximum(u + b_ref[...], 0.0)
        u_ref[...] = u
        s1 = jnp.sum(u, 0, keepdims=True)
        s2 = jnp.sum(u * u, 0, keepdims=True)
        st_ref[...] = jnp.concatenate([s1, s2], 0)[None]

    return pl.pallas_call(
        body,
        grid=(GRID,),
        in_specs=[pl.BlockSpec((2, R, D_IN), lambda i: (0, i, 0)),
                  pl.BlockSpec((R, D_IN), lambda i: (i, 0)),
                  pl.BlockSpec((1, H1), lambda i: (0, 0))],
        out_specs=[pl.BlockSpec((R, H1), lambda i: (i, 0)),
                   pl.BlockSpec((1, 2, H1), lambda i: (i, 0, 0))],
        out_shape=[jax.ShapeDtypeStruct((N, H1), jnp.float32),
                   jax.ShapeDtypeStruct((GRID, 2, H1), jnp.float32)],
    )(p1, ni_b, b1)


def _tc_bn1(u, st1, gamma1, beta1, no_b):
    def body(u_ref, st_ref, g_ref, be_ref, no_ref, h2_ref, t2_ref):
        st = jnp.sum(st_ref[...], 0)
        mean = st[0:1] / N
        var = st[1:2] / N - mean * mean
        h2 = (u_ref[...] - mean) * lax.rsqrt(var + EPS) * g_ref[...] + be_ref[...]
        h2_ref[...] = h2
        t2 = h2 * no_ref[...][:, :H1]
        t2_ref[...] = jnp.concatenate([t2, jnp.zeros((R, D_IN - H1), jnp.float32)], 1)

    return pl.pallas_call(
        body,
        grid=(GRID,),
        in_specs=[pl.BlockSpec((R, H1), lambda i: (i, 0)),
                  pl.BlockSpec((GRID, 2, H1), lambda i: (0, 0, 0)),
                  pl.BlockSpec((1, H1), lambda i: (0, 0)),
                  pl.BlockSpec((1, H1), lambda i: (0, 0)),
                  pl.BlockSpec((R, D_IN), lambda i: (i, 0))],
        out_specs=[pl.BlockSpec((R, H1), lambda i: (i, 0)),
                   pl.BlockSpec((R, D_IN), lambda i: (i, 0))],
        out_shape=[jax.ShapeDtypeStruct((N, H1), jnp.float32),
                   jax.ShapeDtypeStruct((N, D_IN), jnp.float32)],
    )(u, st1, gamma1, beta1, no_b)


def _tc_adj(h2):
    def body(a_ref, b_ref, o_ref):
        p = lax.dot_general(a_ref[...], b_ref[...],
                            (((1,), (1,)), ((), ())),
                            preferred_element_type=jnp.float32,
                    precision=lax.Precision.HIGHEST)
        o_ref[...] = 1.0 / (1.0 + jnp.exp(-p))

    RA = 400
    return pl.pallas_call(
        body,
        grid=(N // RA,),
        in_specs=[pl.BlockSpec((RA, H1), lambda i: (i, 0)),
                  pl.BlockSpec((N, H1), lambda i: (0, 0))],
        out_specs=pl.BlockSpec((RA, N), lambda i: (i, 0)),
        out_shape=jax.ShapeDtypeStruct((N, N), jnp.float32),
    )(h2, h2)


def _tc_rec0(p2, ni_b, Wr0, br0, Wr1, no_b):
    def body(p_ref, ni_ref, w0_ref, b0_ref, w1_ref, no_ref, t3_ref):
        m = (p_ref[0] + p_ref[1])[:, :H1] * ni_ref[...][:, :H1]
        hr = jnp.dot(m, w0_ref[...], preferred_element_type=jnp.float32,
                    precision=lax.Precision.HIGHEST)
        hr = jnp.maximum(hr + b0_ref[...], 0.0)
        t3 = jnp.dot(hr, w1_ref[...], preferred_element_type=jnp.float32,
                    precision=lax.Precision.HIGHEST)
        t3_ref[...] = t3 * no_ref[...]

    return pl.pallas_call(
        body,
        grid=(GRID,),
        in_specs=[pl.BlockSpec((2, R, D_IN), lambda i: (0, i, 0)),
                  pl.BlockSpec((R, D_IN), lambda i: (i, 0)),
                  pl.BlockSpec((H1, H0), lambda i: (0, 0)),
                  pl.BlockSpec((1, H0), lambda i: (0, 0)),
                  pl.BlockSpec((H0, D_IN), lambda i: (0, 0)),
                  pl.BlockSpec((R, D_IN), lambda i: (i, 0))],
        out_specs=pl.BlockSpec((R, D_IN), lambda i: (i, 0)),
        out_shape=jax.ShapeDtypeStruct((N, D_IN), jnp.float32),
    )(p2, ni_b, Wr0, br0, Wr1, no_b)


def _tc_rec1(p3, ni_b, br1):
    def body(p_ref, ni_ref, b_ref, o_ref):
        o_ref[...] = (p_ref[0] + p_ref[1]) * ni_ref[...] + b_ref[...]

    return pl.pallas_call(
        body,
        grid=(GRID,),
        in_specs=[pl.BlockSpec((2, R, D_IN), lambda i: (0, i, 0)),
                  pl.BlockSpec((R, D_IN), lambda i: (i, 0)),
                  pl.BlockSpec((1, D_IN), lambda i: (0, 0))],
        out_specs=pl.BlockSpec((R, D_IN), lambda i: (i, 0)),
        out_shape=jax.ShapeDtypeStruct((N, D_IN), jnp.float32),
    )(p3, ni_b, br1)


def kernel(x, edge_index, W0, b0, gamma0, beta0, W1, b1, gamma1, beta1,
           Wr0, br0, Wr1, br1):
    src = edge_index[0].astype(jnp.int32)
    dst = edge_index[1].astype(jnp.int32)
    npad_e = E2 - E
    trash = N + jnp.arange(npad_e, dtype=jnp.int32) % NTRASH
    src2 = jnp.concatenate([src, trash])
    dst2 = jnp.concatenate([dst, trash])
    z128 = jnp.zeros((RPT, D_IN), jnp.float32)

    deg = _sc_deg()(src2, dst2).reshape(NC, NPAD, 1)
    t0, no_b, ni_b = _tc_norms(deg, x)

    p0 = _sc_agg(D_IN)(t0, src2, dst2, z128)
    h0, st0 = _tc_enc0(p0, ni_b, W0, b0.reshape(1, H0))
    t1 = _tc_bn0_mm1(h0, st0, gamma0.reshape(1, H0), beta0.reshape(1, H0),
                     W1, no_b)

    p1 = _sc_agg(D_IN)(t1, src2, dst2, z128)
    u, st1 = _tc_enc1(p1, ni_b, b1.reshape(1, H1))
    h2, t2 = _tc_bn1(u, st1, gamma1.reshape(1, H1), beta1.reshape(1, H1),
                     no_b)

    adj = _tc_adj(h2)

    p2 = _sc_agg(D_IN)(t2, src2, dst2, z128)
    t3 = _tc_rec0(p2, ni_b, Wr0, br0.reshape(1, H0), Wr1, no_b)

    p3 = _sc_agg(D_IN)(t3, src2, dst2, z128)
    h_out = _tc_rec1(p3, ni_b, br1.reshape(1, D_IN))

    return adj, h_out


# speedup vs baseline: 7.3898x; 1.0298x over previous
import functools

import jax
import jax.numpy as jnp
from jax import lax
from jax.experimental import pallas as pl
from jax.experimental.pallas import tpu as pltpu
import jax.experimental.pallas.tpu_sc as plsc

N = 10000
E = 320000
D_IN = 128
H0 = 256
H1 = 64
EPS = 1e-5

NC = 2
NS = 16
NW = NC * NS
PER_W = E // NW
CH = 80
NCHUNK = PER_W // CH
CH2 = 128
NCHUNK2 = 79
E2 = NW * NCHUNK2 * CH2
PER_W2 = E2 // NW
NTRASH = 96
RPT = 640
NPAD = NS * RPT

R = 1000
GRID = N // R


def _mesh():
    return plsc.VectorSubcoreMesh(core_axis_name="c", subcore_axis_name="s")


@functools.cache
def _sc_deg():
    PER_T = E2 // NS
    NCH_T = PER_T // CH2

    @functools.partial(
        pl.kernel,
        out_type=jax.ShapeDtypeStruct((NC * NPAD,), jnp.float32),
        mesh=_mesh(),
        scratch_types=[
            pltpu.VMEM((3, CH2), jnp.int32),
            pltpu.VMEM((CH2,), jnp.float32),
            pltpu.VMEM((RPT,), jnp.float32),
            pltpu.VMEM_SHARED((NPAD,), jnp.float32),
            pltpu.SemaphoreType.DMA((3,)),
            pltpu.SemaphoreType.DMA((2,)),
        ],
    )
    def deg_kernel(src_hbm, dst_hbm, deg_hbm,
                   idx, ones_v, zbuf, acc, semi, sems):
        c = lax.axis_index("c")
        s = lax.axis_index("s")
        base0 = s * PER_T

        def fill(i, carry):
            ones_v[pl.ds(i * 16, 16)] = jnp.full((16,), 1.0, jnp.float32)
            return carry

        lax.fori_loop(0, CH2 // 16, fill, 0)

        def fill_zero(i, carry):
            zbuf[pl.ds(i * 16, 16)] = jnp.zeros((16,), jnp.float32)
            return carry

        lax.fori_loop(0, RPT // 16, fill_zero, 0)
        pltpu.sync_copy(zbuf, acc.at[pl.ds(s * RPT, RPT)])
        plsc.subcore_barrier()

        def load_idx(j, slot):
            base = pl.multiple_of(base0 + j * CH2, 8)

            @pl.when(c == 0)
            def _():
                pltpu.async_copy(src_hbm.at[pl.ds(base, CH2)], idx.at[slot],
                                 semi.at[slot])

            @pl.when(c == 1)
            def _():
                pltpu.async_copy(dst_hbm.at[pl.ds(base, CH2)], idx.at[slot],
                                 semi.at[slot])

            return pltpu.make_async_copy(
                src_hbm.at[pl.ds(base, CH2)], idx.at[slot], semi.at[slot])

        load_idx(0, 0)

        def body(j, carry):
            b2 = lax.rem(j, 2)
            b3 = lax.rem(j, 3)
            nb3 = lax.rem(j + 1, 3)

            @pl.when(j >= 2)
            def _():
                pltpu.make_async_copy(ones_v, acc.at[idx.at[b3]],
                                      sems.at[b2]).wait()

            @pl.when(j + 1 < NCH_T)
            def _():
                load_idx(j + 1, nb3)

            base = pl.multiple_of(base0 + j * CH2, 8)
            pltpu.make_async_copy(src_hbm.at[pl.ds(base, CH2)], idx.at[b3],
                                  semi.at[b3]).wait()
            pltpu.async_copy(ones_v, acc.at[idx.at[b3]], sems.at[b2],
                             add=True)
            return carry

        lax.fori_loop(0, NCH_T, body, 0)
        pltpu.make_async_copy(ones_v, acc.at[idx.at[0]], sems.at[0]).wait()
        pltpu.make_async_copy(ones_v, acc.at[idx.at[1]], sems.at[1]).wait()
        plsc.subcore_barrier()
        pltpu.sync_copy(acc.at[pl.ds(s * RPT, RPT)],
                        deg_hbm.at[pl.ds(c * NPAD + s * RPT, RPT)])

    return deg_kernel


@functools.cache
def _sc_agg(d):
    @functools.partial(
        pl.kernel,
        out_type=jax.ShapeDtypeStruct((NC, NPAD, d), jnp.float32),
        mesh=_mesh(),
        scratch_types=[
            pltpu.VMEM((3, CH2), jnp.int32),
            pltpu.VMEM((3, CH2), jnp.int32),
            pltpu.VMEM((2, CH2, d), jnp.float32),
            pltpu.VMEM_SHARED((NPAD, d), jnp.float32),
            pltpu.SemaphoreType.DMA((3,)),
            pltpu.SemaphoreType.DMA((3,)),
            pltpu.SemaphoreType.DMA((2,)),
            pltpu.SemaphoreType.DMA((2,)),
        ],
    )
    def agg_kernel(tab_hbm, src_hbm, dst_hbm, zeros_hbm, out_hbm,
                   sidx, didx, rows, acc, semis, semid, semg, sems):
        c = lax.axis_index("c")
        s = lax.axis_index("s")
        wid = s * NC + c
        base0 = wid * PER_W2
        pltpu.sync_copy(zeros_hbm, acc.at[pl.ds(s * RPT, RPT)])
        plsc.subcore_barrier()

        def load_idx(j, slot):
            base = pl.multiple_of(base0 + j * CH2, 8)
            pltpu.async_copy(src_hbm.at[pl.ds(base, CH2)], sidx.at[slot],
                             semis.at[slot])
            pltpu.async_copy(dst_hbm.at[pl.ds(base, CH2)], didx.at[slot],
                             semid.at[slot])

        def wait_idx(j, slot):
            base = pl.multiple_of(base0 + j * CH2, 8)
            pltpu.make_async_copy(src_hbm.at[pl.ds(base, CH2)], sidx.at[slot],
                                  semis.at[slot]).wait()
            pltpu.make_async_copy(dst_hbm.at[pl.ds(base, CH2)], didx.at[slot],
                                  semid.at[slot]).wait()

        load_idx(0, 0)

        def body(j, carry):
            b2 = lax.rem(j, 2)
            o2 = 1 - b2
            b3 = lax.rem(j, 3)
            nb3 = lax.rem(j + 1, 3)

            @pl.when(j >= 2)
            def _():
                pltpu.make_async_copy(rows.at[b2], acc.at[didx.at[nb3]],
                                      sems.at[b2]).wait()

            @pl.when(j + 1 < NCHUNK2)
            def _():
                load_idx(j + 1, nb3)

            @pl.when(j >= 1)
            def _():
                pltpu.make_async_copy(tab_hbm.at[sidx.at[lax.rem(j - 1, 3)]],
                                      rows.at[o2], semg.at[o2]).wait()
                pltpu.async_copy(rows.at[o2], acc.at[didx.at[lax.rem(j - 1, 3)]],
                                 sems.at[o2], add=True)

            wait_idx(j, b3)
            pltpu.async_copy(tab_hbm.at[sidx.at[b3]], rows.at[b2],
                             semg.at[b2])
            return carry

        lax.fori_loop(0, NCHUNK2, body, 0)

        bl2 = (NCHUNK2 - 1) % 2
        bl3 = (NCHUNK2 - 1) % 3
        pltpu.make_async_copy(tab_hbm.at[sidx.at[bl3]], rows.at[bl2],
                              semg.at[bl2]).wait()
        pltpu.async_copy(rows.at[bl2], acc.at[didx.at[bl3]], sems.at[bl2],
                         add=True)
        pltpu.make_async_copy(rows.at[1 - bl2], acc.at[didx.at[bl3]],
                              sems.at[1 - bl2]).wait()
        pltpu.make_async_copy(rows.at[bl2], acc.at[didx.at[bl3]],
                              sems.at[bl2]).wait()
        plsc.subcore_barrier()
        pltpu.sync_copy(acc.at[pl.ds(s * RPT, RPT)],
                        out_hbm.at[c, pl.ds(s * RPT, RPT)])

    return agg_kernel


def _tc_norms(deg, x):
    def body(deg_ref, x_ref, t0_ref, no_ref, ni_ref):
        do = deg_ref[0]
        di = deg_ref[1]
        no = jnp.where(do > 0, lax.rsqrt(do), 0.0)
        ni = jnp.where(di > 0, lax.rsqrt(di), 0.0)
        no_b = jnp.broadcast_to(no, (R, D_IN))
        ni_b = jnp.broadcast_to(ni, (R, D_IN))
        t0_ref[...] = x_ref[...] * no_b
        no_ref[...] = no_b
        ni_ref[...] = ni_b

    return pl.pallas_call(
        body,
        grid=(GRID,),
        in_specs=[pl.BlockSpec((2, R, 1), lambda i: (0, i, 0)),
                  pl.BlockSpec((R, D_IN), lambda i: (i, 0))],
        out_specs=[pl.BlockSpec((R, D_IN), lambda i: (i, 0))] * 3,
        out_shape=[jax.ShapeDtypeStruct((N, D_IN), jnp.float32)] * 3,
    )(deg, x)


def _tc_enc0(p0, ni_b, W0, b0):
    def body(p_ref, ni_ref, w_ref, b_ref, h_ref, st_ref):
        m = (p_ref[0] + p_ref[1]) * ni_ref[...]
        h = jnp.dot(m, w_ref[...], preferred_element_type=jnp.float32,
                    precision=lax.Precision.HIGHEST)
        h = jnp.maximum(h + b_ref[...], 0.0)
        h_ref[...] = h
        s1 = jnp.sum(h, 0, keepdims=True)
        s2 = jnp.sum(h * h, 0, keepdims=True)
        st_ref[...] = jnp.concatenate([s1, s2], 0)[None]

    return pl.pallas_call(
        body,
        grid=(GRID,),
        in_specs=[pl.BlockSpec((2, R, D_IN), lambda i: (0, i, 0)),
                  pl.BlockSpec((R, D_IN), lambda i: (i, 0)),
                  pl.BlockSpec((D_IN, H0), lambda i: (0, 0)),
                  pl.BlockSpec((1, H0), lambda i: (0, 0))],
        out_specs=[pl.BlockSpec((R, H0), lambda i: (i, 0)),
                   pl.BlockSpec((1, 2, H0), lambda i: (i, 0, 0))],
        out_shape=[jax.ShapeDtypeStruct((N, H0), jnp.float32),
                   jax.ShapeDtypeStruct((GRID, 2, H0), jnp.float32)],
    )(p0, ni_b, W0, b0)


def _tc_bn0_mm1(h0, st0, gamma0, beta0, W1, no_b):
    def body(h_ref, st_ref, g_ref, be_ref, w_ref, no_ref, t1_ref):
        st = jnp.sum(st_ref[...], 0)
        mean = st[0:1] / N
        var = st[1:2] / N - mean * mean
        hn = (h_ref[...] - mean) * lax.rsqrt(var + EPS) * g_ref[...] + be_ref[...]
        t1 = jnp.dot(hn, w_ref[...], preferred_element_type=jnp.float32,
                    precision=lax.Precision.HIGHEST)
        t1 = t1 * no_ref[...][:, :H1]
        t1_ref[...] = jnp.concatenate([t1, jnp.zeros((R, D_IN - H1), jnp.float32)], 1)

    return pl.pallas_call(
        body,
        grid=(GRID,),
        in_specs=[pl.BlockSpec((R, H0), lambda i: (i, 0)),
                  pl.BlockSpec((GRID, 2, H0), lambda i: (0, 0, 0)),
                  pl.BlockSpec((1, H0), lambda i: (0, 0)),
                  pl.BlockSpec((1, H0), lambda i: (0, 0)),
                  pl.BlockSpec((H0, H1), lambda i: (0, 0)),
                  pl.BlockSpec((R, D_IN), lambda i: (i, 0))],
        out_specs=pl.BlockSpec((R, D_IN), lambda i: (i, 0)),
        out_shape=jax.ShapeDtypeStruct((N, D_IN), jnp.float32),
    )(h0, st0, gamma0, beta0, W1, no_b)


def _tc_enc1(p1, ni_b, b1):
    def body(p_ref, ni_ref, b_ref, u_ref, st_ref):
        u = (p_ref[0] + p_ref[1])[:, :H1] * ni_ref[...][:, :H1]
        u = jnp.maximum(u + b_ref[...], 0.0)
        u_ref[...] = u
        s1 = jnp.sum(u, 0, keepdims=True)
        s2 = jnp.sum(u * u, 0, keepdims=True)
        st_ref[...] = jnp.concatenate([s1, s2], 0)[None]

    return pl.pallas_call(
        body,
        grid=(GRID,),
        in_specs=[pl.BlockSpec((2, R, D_IN), lambda i: (0, i, 0)),
                  pl.BlockSpec((R, D_IN), lambda i: (i, 0)),
                  pl.BlockSpec((1, H1), lambda i: (0, 0))],
        out_specs=[pl.BlockSpec((R, H1), lambda i: (i, 0)),
                   pl.BlockSpec((1, 2, H1), lambda i: (i, 0, 0))],
        out_shape=[jax.ShapeDtypeStruct((N, H1), jnp.float32),
                   jax.ShapeDtypeStruct((GRID, 2, H1), jnp.float32)],
    )(p1, ni_b, b1)


def _tc_bn1(u, st1, gamma1, beta1, no_b):
    def body(u_ref, st_ref, g_ref, be_ref, no_ref, h2_ref, t2_ref):
        st = jnp.sum(st_ref[...], 0)
        mean = st[0:1] / N
        var = st[1:2] / N - mean * mean
        h2 = (u_ref[...] - mean) * lax.rsqrt(var + EPS) * g_ref[...] + be_ref[...]
        h2_ref[...] = h2
        t2 = h2 * no_ref[...][:, :H1]
        t2_ref[...] = jnp.concatenate([t2, jnp.zeros((R, D_IN - H1), jnp.float32)], 1)

    return pl.pallas_call(
        body,
        grid=(GRID,),
        in_specs=[pl.BlockSpec((R, H1), lambda i: (i, 0)),
                  pl.BlockSpec((GRID, 2, H1), lambda i: (0, 0, 0)),
                  pl.BlockSpec((1, H1), lambda i: (0, 0)),
                  pl.BlockSpec((1, H1), lambda i: (0, 0)),
                  pl.BlockSpec((R, D_IN), lambda i: (i, 0))],
        out_specs=[pl.BlockSpec((R, H1), lambda i: (i, 0)),
                   pl.BlockSpec((R, D_IN), lambda i: (i, 0))],
        out_shape=[jax.ShapeDtypeStruct((N, H1), jnp.float32),
                   jax.ShapeDtypeStruct((N, D_IN), jnp.float32)],
    )(u, st1, gamma1, beta1, no_b)


def _tc_adj(h2):
    RA = 400
    S1 = 4800

    def body(a_ref, b_ref, o_ref):
        p = lax.dot_general(a_ref[...], b_ref[...],
                            (((1,), (1,)), ((), ())),
                            preferred_element_type=jnp.float32,
                            precision=lax.Precision.HIGHEST)
        o_ref[...] = 1.0 / (1.0 + jnp.exp(-p))

    top = pl.pallas_call(
        body,
        grid=(S1 // RA,),
        in_specs=[pl.BlockSpec((RA, H1), lambda i: (i, 0)),
                  pl.BlockSpec((N, H1), lambda i: (0, 0))],
        out_specs=pl.BlockSpec((RA, N), lambda i: (i, 0)),
        out_shape=jax.ShapeDtypeStruct((N, N), jnp.float32),
    )(h2, h2)

    def body2(buf_ref, a_ref, b_ref, o_ref):
        body(a_ref, b_ref, o_ref)

    off = S1 // RA
    return pl.pallas_call(
        body2,
        grid=((N - S1) // RA,),
        in_specs=[pl.BlockSpec(memory_space=pl.ANY),
                  pl.BlockSpec((RA, H1), lambda i: (i + off, 0)),
                  pl.BlockSpec((N, H1), lambda i: (0, 0))],
        out_specs=pl.BlockSpec((RA, N), lambda i: (i + off, 0)),
        out_shape=jax.ShapeDtypeStruct((N, N), jnp.float32),
        input_output_aliases={0: 0},
    )(top, h2, h2)


def _tc_rec0(p2, ni_b, Wr0, br0, Wr1, no_b):
    def body(p_ref, ni_ref, w0_ref, b0_ref, w1_ref, no_ref, t3_ref):
        m = (p_ref[0] + p_ref[1])[:, :H1] * ni_ref[...][:, :H1]
        hr = jnp.dot(m, w0_ref[...], preferred_element_type=jnp.float32,
                    precision=lax.Precision.HIGHEST)
        hr = jnp.maximum(hr + b0_ref[...], 0.0)
        t3 = jnp.dot(hr, w1_ref[...], preferred_element_type=jnp.float32,
                    precision=lax.Precision.HIGHEST)
        t3_ref[...] = t3 * no_ref[...]

    return pl.pallas_call(
        body,
        grid=(GRID,),
        in_specs=[pl.BlockSpec((2, R, D_IN), lambda i: (0, i, 0)),
                  pl.BlockSpec((R, D_IN), lambda i: (i, 0)),
                  pl.BlockSpec((H1, H0), lambda i: (0, 0)),
                  pl.BlockSpec((1, H0), lambda i: (0, 0)),
                  pl.BlockSpec((H0, D_IN), lambda i: (0, 0)),
                  pl.BlockSpec((R, D_IN), lambda i: (i, 0))],
        out_specs=pl.BlockSpec((R, D_IN), lambda i: (i, 0)),
        out_shape=jax.ShapeDtypeStruct((N, D_IN), jnp.float32),
    )(p2, ni_b, Wr0, br0, Wr1, no_b)


def _tc_rec1(p3, ni_b, br1):
    def body(p_ref, ni_ref, b_ref, o_ref):
        o_ref[...] = (p_ref[0] + p_ref[1]) * ni_ref[...] + b_ref[...]

    return pl.pallas_call(
        body,
        grid=(GRID,),
        in_specs=[pl.BlockSpec((2, R, D_IN), lambda i: (0, i, 0)),
                  pl.BlockSpec((R, D_IN), lambda i: (i, 0)),
                  pl.BlockSpec((1, D_IN), lambda i: (0, 0))],
        out_specs=pl.BlockSpec((R, D_IN), lambda i: (i, 0)),
        out_shape=jax.ShapeDtypeStruct((N, D_IN), jnp.float32),
    )(p3, ni_b, br1)


def kernel(x, edge_index, W0, b0, gamma0, beta0, W1, b1, gamma1, beta1,
           Wr0, br0, Wr1, br1):
    src = edge_index[0].astype(jnp.int32)
    dst = edge_index[1].astype(jnp.int32)
    npad_e = E2 - E
    trash = N + jnp.arange(npad_e, dtype=jnp.int32) % NTRASH
    src2 = jnp.concatenate([src, trash])
    dst2 = jnp.concatenate([dst, trash])
    z128 = jnp.zeros((RPT, D_IN), jnp.float32)

    deg = _sc_deg()(src2, dst2).reshape(NC, NPAD, 1)
    t0, no_b, ni_b = _tc_norms(deg, x)

    p0 = _sc_agg(D_IN)(t0, src2, dst2, z128)
    h0, st0 = _tc_enc0(p0, ni_b, W0, b0.reshape(1, H0))
    t1 = _tc_bn0_mm1(h0, st0, gamma0.reshape(1, H0), beta0.reshape(1, H0),
                     W1, no_b)

    p1 = _sc_agg(D_IN)(t1, src2, dst2, z128)
    u, st1 = _tc_enc1(p1, ni_b, b1.reshape(1, H1))
    h2, t2 = _tc_bn1(u, st1, gamma1.reshape(1, H1), beta1.reshape(1, H1),
                     no_b)

    adj = _tc_adj(h2)

    p2 = _sc_agg(D_IN)(t2, src2, dst2, z128)
    t3 = _tc_rec0(p2, ni_b, Wr0, br0.reshape(1, H0), Wr1, no_b)

    p3 = _sc_agg(D_IN)(t3, src2, dst2, z128)
    h_out = _tc_rec1(p3, ni_b, br1.reshape(1, D_IN))

    return adj, h_out


# speedup vs baseline: 8.8320x; 1.1952x over previous
import functools

import jax
import jax.numpy as jnp
from jax import lax
from jax.experimental import pallas as pl
from jax.experimental.pallas import tpu as pltpu
import jax.experimental.pallas.tpu_sc as plsc

N = 10000
E = 320000
D_IN = 128
H0 = 256
H1 = 64
EPS = 1e-5

NC = 2
NS = 16
NW = NC * NS
PER_W = E // NW
CH = 80
NCHUNK = PER_W // CH
CH2 = 128
NCHUNK2 = 79
E2 = NW * NCHUNK2 * CH2
PER_W2 = E2 // NW
NTRASH = 96
RPT = 640
NPAD = NS * RPT

R = 1000
GRID = N // R


def _mesh():
    return plsc.VectorSubcoreMesh(core_axis_name="c", subcore_axis_name="s")


@functools.cache
def _sc_deg():
    PER_T = E2 // NS
    NCH_T = PER_T // CH2

    @functools.partial(
        pl.kernel,
        out_type=jax.ShapeDtypeStruct((NC * NPAD,), jnp.float32),
        mesh=_mesh(),
        scratch_types=[
            pltpu.VMEM((3, CH2), jnp.int32),
            pltpu.VMEM((CH2,), jnp.float32),
            pltpu.VMEM((RPT,), jnp.float32),
            pltpu.VMEM_SHARED((NPAD,), jnp.float32),
            pltpu.SemaphoreType.DMA((3,)),
            pltpu.SemaphoreType.DMA((2,)),
        ],
    )
    def deg_kernel(src_hbm, dst_hbm, deg_hbm,
                   idx, ones_v, zbuf, acc, semi, sems):
        c = lax.axis_index("c")
        s = lax.axis_index("s")
        base0 = s * PER_T

        def fill(i, carry):
            ones_v[pl.ds(i * 16, 16)] = jnp.full((16,), 1.0, jnp.float32)
            return carry

        lax.fori_loop(0, CH2 // 16, fill, 0)

        def fill_zero(i, carry):
            zbuf[pl.ds(i * 16, 16)] = jnp.zeros((16,), jnp.float32)
            return carry

        lax.fori_loop(0, RPT // 16, fill_zero, 0)
        pltpu.sync_copy(zbuf, acc.at[pl.ds(s * RPT, RPT)])
        plsc.subcore_barrier()

        def load_idx(j, slot):
            base = pl.multiple_of(base0 + j * CH2, 8)

            @pl.when(c == 0)
            def _():
                pltpu.async_copy(src_hbm.at[pl.ds(base, CH2)], idx.at[slot],
                                 semi.at[slot])

            @pl.when(c == 1)
            def _():
                pltpu.async_copy(dst_hbm.at[pl.ds(base, CH2)], idx.at[slot],
                                 semi.at[slot])

            return pltpu.make_async_copy(
                src_hbm.at[pl.ds(base, CH2)], idx.at[slot], semi.at[slot])

        load_idx(0, 0)

        def body(j, carry):
            b2 = lax.rem(j, 2)
            b3 = lax.rem(j, 3)
            nb3 = lax.rem(j + 1, 3)

            @pl.when(j >= 2)
            def _():
                pltpu.make_async_copy(ones_v, acc.at[idx.at[b3]],
                                      sems.at[b2]).wait()

            @pl.when(j + 1 < NCH_T)
            def _():
                load_idx(j + 1, nb3)

            base = pl.multiple_of(base0 + j * CH2, 8)
            pltpu.make_async_copy(src_hbm.at[pl.ds(base, CH2)], idx.at[b3],
                                  semi.at[b3]).wait()
            pltpu.async_copy(ones_v, acc.at[idx.at[b3]], sems.at[b2],
                             add=True)
            return carry

        lax.fori_loop(0, NCH_T, body, 0)
        pltpu.make_async_copy(ones_v, acc.at[idx.at[0]], sems.at[0]).wait()
        pltpu.make_async_copy(ones_v, acc.at[idx.at[1]], sems.at[1]).wait()
        plsc.subcore_barrier()
        pltpu.sync_copy(acc.at[pl.ds(s * RPT, RPT)],
                        deg_hbm.at[pl.ds(c * NPAD + s * RPT, RPT)])

    return deg_kernel


@functools.cache
def _sc_agg(d):
    @functools.partial(
        pl.kernel,
        out_type=jax.ShapeDtypeStruct((NC, NPAD, d), jnp.float32),
        mesh=_mesh(),
        scratch_types=[
            pltpu.VMEM((3, CH2), jnp.int32),
            pltpu.VMEM((3, CH2), jnp.int32),
            pltpu.VMEM((2, CH2, d), jnp.float32),
            pltpu.VMEM_SHARED((NPAD, d), jnp.float32),
            pltpu.SemaphoreType.DMA((3,)),
            pltpu.SemaphoreType.DMA((3,)),
            pltpu.SemaphoreType.DMA((2,)),
            pltpu.SemaphoreType.DMA((2,)),
        ],
    )
    def agg_kernel(tab_hbm, src_hbm, dst_hbm, zeros_hbm, out_hbm,
                   sidx, didx, rows, acc, semis, semid, semg, sems):
        c = lax.axis_index("c")
        s = lax.axis_index("s")
        wid = s * NC + c
        base0 = wid * PER_W2
        pltpu.sync_copy(zeros_hbm, acc.at[pl.ds(s * RPT, RPT)])
        plsc.subcore_barrier()

        def load_idx(j, slot):
            base = pl.multiple_of(base0 + j * CH2, 8)
            pltpu.async_copy(src_hbm.at[pl.ds(base, CH2)], sidx.at[slot],
                             semis.at[slot])
            pltpu.async_copy(dst_hbm.at[pl.ds(base, CH2)], didx.at[slot],
                             semid.at[slot])

        def wait_idx(j, slot):
            base = pl.multiple_of(base0 + j * CH2, 8)
            pltpu.make_async_copy(src_hbm.at[pl.ds(base, CH2)], sidx.at[slot],
                                  semis.at[slot]).wait()
            pltpu.make_async_copy(dst_hbm.at[pl.ds(base, CH2)], didx.at[slot],
                                  semid.at[slot]).wait()

        load_idx(0, 0)

        def body(j, carry):
            b2 = lax.rem(j, 2)
            o2 = 1 - b2
            b3 = lax.rem(j, 3)
            nb3 = lax.rem(j + 1, 3)

            @pl.when(j >= 2)
            def _():
                pltpu.make_async_copy(rows.at[b2], acc.at[didx.at[nb3]],
                                      sems.at[b2]).wait()

            @pl.when(j + 1 < NCHUNK2)
            def _():
                load_idx(j + 1, nb3)

            @pl.when(j >= 1)
            def _():
                pltpu.make_async_copy(tab_hbm.at[sidx.at[lax.rem(j - 1, 3)]],
                                      rows.at[o2], semg.at[o2]).wait()
                pltpu.async_copy(rows.at[o2], acc.at[didx.at[lax.rem(j - 1, 3)]],
                                 sems.at[o2], add=True)

            wait_idx(j, b3)
            pltpu.async_copy(tab_hbm.at[sidx.at[b3]], rows.at[b2],
                             semg.at[b2])
            return carry

        lax.fori_loop(0, NCHUNK2, body, 0)

        bl2 = (NCHUNK2 - 1) % 2
        bl3 = (NCHUNK2 - 1) % 3
        pltpu.make_async_copy(tab_hbm.at[sidx.at[bl3]], rows.at[bl2],
                              semg.at[bl2]).wait()
        pltpu.async_copy(rows.at[bl2], acc.at[didx.at[bl3]], sems.at[bl2],
                         add=True)
        pltpu.make_async_copy(rows.at[1 - bl2], acc.at[didx.at[bl3]],
                              sems.at[1 - bl2]).wait()
        pltpu.make_async_copy(rows.at[bl2], acc.at[didx.at[bl3]],
                              sems.at[bl2]).wait()
        plsc.subcore_barrier()
        pltpu.sync_copy(acc.at[pl.ds(s * RPT, RPT)],
                        out_hbm.at[c, pl.ds(s * RPT, RPT)])

    return agg_kernel


def _tc_norms(deg, x):
    def body(deg_ref, x_ref, t0_ref, no_ref, ni_ref):
        do = deg_ref[0]
        di = deg_ref[1]
        no = jnp.where(do > 0, lax.rsqrt(do), 0.0)
        ni = jnp.where(di > 0, lax.rsqrt(di), 0.0)
        no_b = jnp.broadcast_to(no, (R, D_IN))
        ni_b = jnp.broadcast_to(ni, (R, D_IN))
        t0_ref[...] = x_ref[...] * no_b
        no_ref[...] = no_b
        ni_ref[...] = ni_b

    return pl.pallas_call(
        body,
        grid=(GRID,),
        in_specs=[pl.BlockSpec((2, R, 1), lambda i: (0, i, 0)),
                  pl.BlockSpec((R, D_IN), lambda i: (i, 0))],
        out_specs=[pl.BlockSpec((R, D_IN), lambda i: (i, 0))] * 3,
        out_shape=[jax.ShapeDtypeStruct((N, D_IN), jnp.float32)] * 3,
    )(deg, x)


def _tc_enc0(p0, ni_b, W0, b0):
    def body(p_ref, ni_ref, w_ref, b_ref, h_ref, st_ref):
        m = (p_ref[0] + p_ref[1]) * ni_ref[...]
        h = jnp.dot(m, w_ref[...], preferred_element_type=jnp.float32,
                    precision=lax.Precision.HIGHEST)
        h = jnp.maximum(h + b_ref[...], 0.0)
        h_ref[...] = h
        s1 = jnp.sum(h, 0, keepdims=True)
        s2 = jnp.sum(h * h, 0, keepdims=True)
        st_ref[...] = jnp.concatenate([s1, s2], 0)[None]

    return pl.pallas_call(
        body,
        grid=(GRID,),
        in_specs=[pl.BlockSpec((2, R, D_IN), lambda i: (0, i, 0)),
                  pl.BlockSpec((R, D_IN), lambda i: (i, 0)),
                  pl.BlockSpec((D_IN, H0), lambda i: (0, 0)),
                  pl.BlockSpec((1, H0), lambda i: (0, 0))],
        out_specs=[pl.BlockSpec((R, H0), lambda i: (i, 0)),
                   pl.BlockSpec((1, 2, H0), lambda i: (i, 0, 0))],
        out_shape=[jax.ShapeDtypeStruct((N, H0), jnp.float32),
                   jax.ShapeDtypeStruct((GRID, 2, H0), jnp.float32)],
    )(p0, ni_b, W0, b0)


def _tc_bn0_mm1(h0, st0, gamma0, beta0, W1, no_b):
    def body(h_ref, st_ref, g_ref, be_ref, w_ref, no_ref, t1_ref):
        st = jnp.sum(st_ref[...], 0)
        mean = st[0:1] / N
        var = st[1:2] / N - mean * mean
        hn = (h_ref[...] - mean) * lax.rsqrt(var + EPS) * g_ref[...] + be_ref[...]
        t1 = jnp.dot(hn, w_ref[...], preferred_element_type=jnp.float32,
                    precision=lax.Precision.HIGHEST)
        t1 = t1 * no_ref[...][:, :H1]
        t1_ref[...] = jnp.concatenate([t1, jnp.zeros((R, D_IN - H1), jnp.float32)], 1)

    return pl.pallas_call(
        body,
        grid=(GRID,),
        in_specs=[pl.BlockSpec((R, H0), lambda i: (i, 0)),
                  pl.BlockSpec((GRID, 2, H0), lambda i: (0, 0, 0)),
                  pl.BlockSpec((1, H0), lambda i: (0, 0)),
                  pl.BlockSpec((1, H0), lambda i: (0, 0)),
                  pl.BlockSpec((H0, H1), lambda i: (0, 0)),
                  pl.BlockSpec((R, D_IN), lambda i: (i, 0))],
        out_specs=pl.BlockSpec((R, D_IN), lambda i: (i, 0)),
        out_shape=jax.ShapeDtypeStruct((N, D_IN), jnp.float32),
    )(h0, st0, gamma0, beta0, W1, no_b)


def _tc_enc1(p1, ni_b, b1):
    def body(p_ref, ni_ref, b_ref, u_ref, st_ref):
        u = (p_ref[0] + p_ref[1])[:, :H1] * ni_ref[...][:, :H1]
        u = jnp.maximum(u + b_ref[...], 0.0)
        u_ref[...] = u
        s1 = jnp.sum(u, 0, keepdims=True)
        s2 = jnp.sum(u * u, 0, keepdims=True)
        st_ref[...] = jnp.concatenate([s1, s2], 0)[None]

    return pl.pallas_call(
        body,
        grid=(GRID,),
        in_specs=[pl.BlockSpec((2, R, D_IN), lambda i: (0, i, 0)),
                  pl.BlockSpec((R, D_IN), lambda i: (i, 0)),
                  pl.BlockSpec((1, H1), lambda i: (0, 0))],
        out_specs=[pl.BlockSpec((R, H1), lambda i: (i, 0)),
                   pl.BlockSpec((1, 2, H1), lambda i: (i, 0, 0))],
        out_shape=[jax.ShapeDtypeStruct((N, H1), jnp.float32),
                   jax.ShapeDtypeStruct((GRID, 2, H1), jnp.float32)],
    )(p1, ni_b, b1)


def _tc_bn1(u, st1, gamma1, beta1, no_b):
    def body(u_ref, st_ref, g_ref, be_ref, no_ref, h2_ref, t2_ref):
        st = jnp.sum(st_ref[...], 0)
        mean = st[0:1] / N
        var = st[1:2] / N - mean * mean
        h2 = (u_ref[...] - mean) * lax.rsqrt(var + EPS) * g_ref[...] + be_ref[...]
        h2_ref[...] = h2
        t2 = h2 * no_ref[...][:, :H1]
        t2_ref[...] = jnp.concatenate([t2, jnp.zeros((R, D_IN - H1), jnp.float32)], 1)

    return pl.pallas_call(
        body,
        grid=(GRID,),
        in_specs=[pl.BlockSpec((R, H1), lambda i: (i, 0)),
                  pl.BlockSpec((GRID, 2, H1), lambda i: (0, 0, 0)),
                  pl.BlockSpec((1, H1), lambda i: (0, 0)),
                  pl.BlockSpec((1, H1), lambda i: (0, 0)),
                  pl.BlockSpec((R, D_IN), lambda i: (i, 0))],
        out_specs=[pl.BlockSpec((R, H1), lambda i: (i, 0)),
                   pl.BlockSpec((R, D_IN), lambda i: (i, 0))],
        out_shape=[jax.ShapeDtypeStruct((N, H1), jnp.float32),
                   jax.ShapeDtypeStruct((N, D_IN), jnp.float32)],
    )(u, st1, gamma1, beta1, no_b)


def _tc_adj(h2):
    RA = 400
    S1 = 4800

    def body(a_ref, b_ref, o_ref):
        p = lax.dot_general(a_ref[...], b_ref[...],
                            (((1,), (1,)), ((), ())),
                            preferred_element_type=jnp.float32,
                            precision=lax.Precision.HIGHEST)
        o_ref[...] = 1.0 / (1.0 + jnp.exp(-p))

    def body_top(a_ref, b_ref, o_ref, tok_ref):
        body(a_ref, b_ref, o_ref)
        tok_ref[...] = a_ref[:8, :]

    top, tok = pl.pallas_call(
        body_top,
        grid=(S1 // RA,),
        in_specs=[pl.BlockSpec((RA, H1), lambda i: (i, 0)),
                  pl.BlockSpec((N, H1), lambda i: (0, 0))],
        out_specs=[pl.BlockSpec((RA, N), lambda i: (i, 0)),
                   pl.BlockSpec((8, H1), lambda i: (0, 0))],
        out_shape=[jax.ShapeDtypeStruct((N, N), jnp.float32),
                   jax.ShapeDtypeStruct((8, H1), jnp.float32)],
    )(h2, h2)

    def body2(buf_ref, a_ref, b_ref, o_ref):
        body(a_ref, b_ref, o_ref)

    off = S1 // RA
    adj = pl.pallas_call(
        body2,
        grid=((N - S1) // RA,),
        in_specs=[pl.BlockSpec(memory_space=pl.ANY),
                  pl.BlockSpec((RA, H1), lambda i: (i + off, 0)),
                  pl.BlockSpec((N, H1), lambda i: (0, 0))],
        out_specs=pl.BlockSpec((RA, N), lambda i: (i + off, 0)),
        out_shape=jax.ShapeDtypeStruct((N, N), jnp.float32),
        input_output_aliases={0: 0},
    )(top, h2, h2)
    return adj, tok


def _tc_rec0(p2, ni_b, Wr0, br0, Wr1, no_b, tok):
    def body(p_ref, ni_ref, w0_ref, b0_ref, w1_ref, no_ref, tok_ref, t3_ref):
        m = (p_ref[0] + p_ref[1])[:, :H1] * ni_ref[...][:, :H1]
        hr = jnp.dot(m, w0_ref[...], preferred_element_type=jnp.float32)
        hr = jnp.maximum(hr + b0_ref[...], 0.0)
        t3 = jnp.dot(hr, w1_ref[...], preferred_element_type=jnp.float32)
        t3_ref[...] = t3 * no_ref[...]

    return pl.pallas_call(
        body,
        grid=(GRID,),
        in_specs=[pl.BlockSpec((2, R, D_IN), lambda i: (0, i, 0)),
                  pl.BlockSpec((R, D_IN), lambda i: (i, 0)),
                  pl.BlockSpec((H1, H0), lambda i: (0, 0)),
                  pl.BlockSpec((1, H0), lambda i: (0, 0)),
                  pl.BlockSpec((H0, D_IN), lambda i: (0, 0)),
                  pl.BlockSpec((R, D_IN), lambda i: (i, 0)),
                  pl.BlockSpec((8, H1), lambda i: (0, 0))],
        out_specs=pl.BlockSpec((R, D_IN), lambda i: (i, 0)),
        out_shape=jax.ShapeDtypeStruct((N, D_IN), jnp.float32),
    )(p2, ni_b, Wr0, br0, Wr1, no_b, tok)


def _tc_rec1(p3, ni_b, br1):
    def body(p_ref, ni_ref, b_ref, o_ref):
        o_ref[...] = (p_ref[0] + p_ref[1]) * ni_ref[...] + b_ref[...]

    return pl.pallas_call(
        body,
        grid=(GRID,),
        in_specs=[pl.BlockSpec((2, R, D_IN), lambda i: (0, i, 0)),
                  pl.BlockSpec((R, D_IN), lambda i: (i, 0)),
                  pl.BlockSpec((1, D_IN), lambda i: (0, 0))],
        out_specs=pl.BlockSpec((R, D_IN), lambda i: (i, 0)),
        out_shape=jax.ShapeDtypeStruct((N, D_IN), jnp.float32),
    )(p3, ni_b, br1)


def kernel(x, edge_index, W0, b0, gamma0, beta0, W1, b1, gamma1, beta1,
           Wr0, br0, Wr1, br1):
    src = edge_index[0].astype(jnp.int32)
    dst = edge_index[1].astype(jnp.int32)
    npad_e = E2 - E
    trash = N + jnp.arange(npad_e, dtype=jnp.int32) % NTRASH
    src2 = jnp.concatenate([src, trash])
    dst2 = jnp.concatenate([dst, trash])
    z128 = jnp.zeros((RPT, D_IN), jnp.float32)

    deg = _sc_deg()(src2, dst2).reshape(NC, NPAD, 1)
    t0, no_b, ni_b = _tc_norms(deg, x)

    p0 = _sc_agg(D_IN)(t0, src2, dst2, z128)
    h0, st0 = _tc_enc0(p0, ni_b, W0, b0.reshape(1, H0))
    t1 = _tc_bn0_mm1(h0, st0, gamma0.reshape(1, H0), beta0.reshape(1, H0),
                     W1, no_b)

    p1 = _sc_agg(D_IN)(t1, src2, dst2, z128)
    u, st1 = _tc_enc1(p1, ni_b, b1.reshape(1, H1))
    h2, t2 = _tc_bn1(u, st1, gamma1.reshape(1, H1), beta1.reshape(1, H1),
                     no_b)

    adj, tok = _tc_adj(h2)

    p2 = _sc_agg(D_IN)(t2, src2, dst2, z128)
    t3 = _tc_rec0(p2, ni_b, Wr0, br0.reshape(1, H0), Wr1, no_b, tok)

    p3 = _sc_agg(D_IN)(t3, src2, dst2, z128)
    h_out = _tc_rec1(p3, ni_b, br1.reshape(1, D_IN))

    return adj, h_out
